# baseline (speedup 1.0000x reference)
import jax
import jax.numpy as jnp
from jax.experimental import pallas as pl
from jax.experimental.pallas import tpu as pltpu

_BM = 448
_BMB = 1024


def _layer1_kernel(adj_ref, x_ref, w1_ref, b1_ref, w2_ref, s2_ref, q_ref,
                   s1_scr):
    @pl.when(pl.program_id(0) == 0)
    def _():
        s1_scr[...] = jnp.dot(x_ref[...], w1_ref[...],
                              preferred_element_type=jnp.float32)

    a = adj_ref[...]
    h = jnp.dot(a, s1_scr[...], preferred_element_type=jnp.float32) + b1_ref[...]
    h = jnp.maximum(h, 0.0)
    s2 = jnp.dot(h, w2_ref[...], preferred_element_type=jnp.float32)
    s2_ref[...] = s2.astype(jnp.float8_e4m3fn)
    q_ref[...] = a.astype(jnp.float8_e4m3fn)


def _layer2_kernel(q_ref, s2_ref, b2_ref, out_ref):
    acc = jax.lax.dot_general(
        q_ref[...], s2_ref[...],
        dimension_numbers=(((1,), (0,)), ((), ())),
        preferred_element_type=jnp.float32)
    o = acc + b2_ref[...]
    out_ref[...] = jnp.maximum(o, 0.0)


def kernel(x, adj, W1, b1, W2, b2):
    n, nfeat = x.shape
    hidden = W1.shape[1]
    nclass = W2.shape[1]
    b1r = b1.reshape(1, hidden)
    b2r = b2.reshape(1, nclass)

    grid = (pl.cdiv(n, _BM),)
    s2, q = pl.pallas_call(
        _layer1_kernel,
        grid=grid,
        in_specs=[
            pl.BlockSpec((_BM, n), lambda i: (i, 0)),
            pl.BlockSpec((n, nfeat), lambda i: (0, 0)),
            pl.BlockSpec((nfeat, hidden), lambda i: (0, 0)),
            pl.BlockSpec((1, hidden), lambda i: (0, 0)),
            pl.BlockSpec((hidden, nclass), lambda i: (0, 0)),
        ],
        out_specs=[
            pl.BlockSpec((_BM, nclass), lambda i: (i, 0)),
            pl.BlockSpec((_BM, n), lambda i: (i, 0)),
        ],
        out_shape=[
            jax.ShapeDtypeStruct((n, nclass), jnp.float8_e4m3fn),
            jax.ShapeDtypeStruct((n, n), jnp.float8_e4m3fn),
        ],
        scratch_shapes=[pltpu.VMEM((n, hidden), jnp.float32)],
        compiler_params=pltpu.CompilerParams(
            dimension_semantics=("arbitrary",),
        ),
    )(adj, x, W1, b1r, W2)

    grid_b = (pl.cdiv(n, _BMB),)
    out = pl.pallas_call(
        _layer2_kernel,
        grid=grid_b,
        in_specs=[
            pl.BlockSpec((_BMB, n), lambda i: (i, 0)),
            pl.BlockSpec((n, nclass), lambda i: (0, 0)),
            pl.BlockSpec((1, nclass), lambda i: (0, 0)),
        ],
        out_specs=pl.BlockSpec((_BMB, nclass), lambda i: (i, 0)),
        out_shape=jax.ShapeDtypeStruct((n, nclass), jnp.float32),
        compiler_params=pltpu.CompilerParams(
            dimension_semantics=("parallel",),
        ),
    )(q, s2, b2r)
    return out
